# Initial kernel scaffold; baseline (speedup 1.0000x reference)
import jax
import jax.numpy as jnp
from jax.experimental import pallas as pl


def kernel(local_weight, keyframe_poses, keyframe_map, idx):
    raise NotImplementedError("write your pallas kernel here")


# speedup vs baseline: 1.1414x; 1.1414x over previous
import functools
import math

import jax
import jax.numpy as jnp
from jax import lax
from jax.experimental import pallas as pl
from jax.experimental.pallas import tpu as pltpu
from jax.experimental.pallas import tpu_sc as plsc

NC, NS, L = 2, 16, 16
NW = NC * NS
IDXW = 128

_A_COEF = tuple((-1.0) ** i / math.factorial(2 * i + 1) for i in range(11))
_B_COEF = tuple((-1.0) ** i / math.factorial(2 * i + 2) for i in range(11))
_C_COEF = tuple((-1.0) ** i / math.factorial(2 * i + 3) for i in range(11))


def _horner(coefs, x2):
    acc = jnp.full((L,), coefs[-1], jnp.float32)
    for c in reversed(coefs[:-1]):
        acc = acc * x2 + jnp.full((L,), c, jnp.float32)
    return acc


@functools.lru_cache(maxsize=None)
def _make_sc_kernel(n, b):
    b_per_w = b // NW
    n_sub = b_per_w // IDXW
    n_chunk = b_per_w // L
    mesh = plsc.VectorSubcoreMesh(core_axis_name="c", subcore_axis_name="s")

    @functools.partial(
        pl.kernel,
        out_type=jax.ShapeDtypeStruct((b * 12,), jnp.float32),
        mesh=mesh,
        scratch_types=[
            pltpu.VMEM((n_sub, IDXW), jnp.int32),
            pltpu.VMEM((n_sub, IDXW), jnp.int32),
            pltpu.VMEM((b_per_w, 8), jnp.float32),
            pltpu.VMEM((b_per_w, 16), jnp.float32),
            pltpu.VMEM((b_per_w * 12,), jnp.float32),
            pltpu.SemaphoreType.DMA,
            pltpu.SemaphoreType.DMA,
            pltpu.SemaphoreType.DMA,
        ],
        compiler_params=pltpu.CompilerParams(
            needs_layout_passes=False, use_tc_tiling_on_sc=False),
    )
    def sc(lw_hbm, kfp_hbm, map_hbm, idx_hbm, out_hbm,
           idx_v, kfi_v, wu_v, kf_v, out_v, sem_w, sem_m, sem_k):
        wid = lax.axis_index("s") * NC + lax.axis_index("c")
        base = wid * b_per_w
        pltpu.sync_copy(idx_hbm.at[pl.ds(wid * n_sub, n_sub)], idx_v)

        cps_w, cps_m = [], []
        for j in range(n_sub):
            cps_w.append(pltpu.async_copy(
                lw_hbm.at[idx_v.at[j]], wu_v.at[pl.ds(j * IDXW, IDXW)], sem_w))
            cps_m.append(pltpu.async_copy(
                map_hbm.at[idx_v.at[j]], kfi_v.at[j], sem_m))
        for c in cps_m:
            c.wait()
        cps_k = []
        for j in range(n_sub):
            cps_k.append(pltpu.async_copy(
                kfp_hbm.at[kfi_v.at[j]], kf_v.at[pl.ds(j * IDXW, IDXW)], sem_k))
        for c in cps_w:
            c.wait()
        for c in cps_k:
            c.wait()

        iota = lax.iota(jnp.int32, L)
        one = jnp.full((L,), 1.0, jnp.float32)

        def body(i, carry):
            rows = iota + i * L

            def wcol(c):
                return plsc.load_gather(
                    wu_v, [rows, jnp.full((L,), c, jnp.int32)])

            w0, w1, w2 = wcol(0), wcol(1), wcol(2)
            u0, u1, u2 = wcol(3), wcol(4), wcol(5)
            t2 = w0 * w0 + w1 * w1 + w2 * w2
            A = _horner(_A_COEF, t2)
            Bc = _horner(_B_COEF, t2)
            C = _horner(_C_COEF, t2)
            w00, w11, w22 = w0 * w0, w1 * w1, w2 * w2
            w01, w02, w12 = w0 * w1, w0 * w2, w1 * w2
            r00 = one + Bc * (w00 - t2)
            r01 = Bc * w01 - A * w2
            r02 = Bc * w02 + A * w1
            r10 = Bc * w01 + A * w2
            r11 = one + Bc * (w11 - t2)
            r12 = Bc * w12 - A * w0
            r20 = Bc * w02 - A * w1
            r21 = Bc * w12 + A * w0
            r22 = one + Bc * (w22 - t2)
            v00 = one + C * (w00 - t2)
            v01 = C * w01 - Bc * w2
            v02 = C * w02 + Bc * w1
            v10 = C * w01 + Bc * w2
            v11 = one + C * (w11 - t2)
            v12 = C * w12 - Bc * w0
            v20 = C * w02 - Bc * w1
            v21 = C * w12 + Bc * w0
            v22 = one + C * (w22 - t2)
            ta0 = v00 * u0 + v01 * u1 + v02 * u2
            ta1 = v10 * u0 + v11 * u1 + v12 * u2
            ta2 = v20 * u0 + v21 * u1 + v22 * u2

            def kcol(c):
                return plsc.load_gather(
                    kf_v, [rows, jnp.full((L,), c, jnp.int32)])

            b00, b01, b02, tb0 = kcol(0), kcol(1), kcol(2), kcol(3)
            b10, b11, b12, tb1 = kcol(4), kcol(5), kcol(6), kcol(7)
            b20, b21, b22, tb2 = kcol(8), kcol(9), kcol(10), kcol(11)

            outs = (
                b00 * r00 + b01 * r10 + b02 * r20,
                b00 * r01 + b01 * r11 + b02 * r21,
                b00 * r02 + b01 * r12 + b02 * r22,
                b00 * ta0 + b01 * ta1 + b02 * ta2 + tb0,
                b10 * r00 + b11 * r10 + b12 * r20,
                b10 * r01 + b11 * r11 + b12 * r21,
                b10 * r02 + b11 * r12 + b12 * r22,
                b10 * ta0 + b11 * ta1 + b12 * ta2 + tb1,
                b20 * r00 + b21 * r10 + b22 * r20,
                b20 * r01 + b21 * r11 + b22 * r21,
                b20 * r02 + b21 * r12 + b22 * r22,
                b20 * ta0 + b21 * ta1 + b22 * ta2 + tb2,
            )
            flat = rows * 12
            for c, val in enumerate(outs):
                plsc.store_scatter(out_v, [flat + c], val)
            return carry

        lax.fori_loop(0, n_chunk, body, 0)
        pltpu.sync_copy(out_v, out_hbm.at[pl.ds(base * 12, b_per_w * 12)])

    return sc


def kernel(local_weight, keyframe_poses, keyframe_map, idx):
    n = local_weight.shape[0]
    b = idx.shape[0]
    lw8 = jnp.pad(local_weight, ((0, 0), (0, 2)))
    kf16 = jnp.pad(keyframe_poses.reshape(n, 12), ((0, 0), (0, 4)))
    idx2 = idx.reshape(b // IDXW, IDXW)
    out = _make_sc_kernel(n, b)(lw8, kf16, keyframe_map, idx2)
    return out.reshape(b, 3, 4)
